# baseline (speedup 1.0000x reference)
import jax
import jax.numpy as jnp
from jax.experimental import pallas as pl
from jax.experimental.pallas import tpu as pltpu

_N = 4096
_DIN = 128
_DOUT = 128
_BLK = 256
_NBLK = _N // _BLK


def _fused(a_ref, x_ref, w_ref, b_ref, out_ref,
           abf_ref, degb_ref, h_ref, hs_ref, acc_ref):
    t = pl.program_id(0)
    rows = pl.ds(t * _BLK, _BLK)
    cols = pl.ds(t * _BLK, _BLK)

    @pl.when(t == 0)
    def _init():
        h_ref[...] = jnp.dot(x_ref[...], w_ref[...],
                             preferred_element_type=jnp.float32)
        hs_ref[...] = jnp.zeros((_N, _DOUT), jnp.bfloat16)

    a = a_ref[...]
    s = jnp.sum(a, axis=1, keepdims=True)
    d = jax.lax.rsqrt(s)
    d = jnp.where(jnp.isinf(d), 0.0, d)
    db = jnp.broadcast_to(d, (_BLK, _DOUT))
    degb_ref[rows, :] = db
    hs_blk = (db * h_ref[rows, :]).astype(jnp.bfloat16)
    hs_ref[rows, :] = hs_blk

    acc_ref[...] += jnp.dot(abf_ref[:, cols], hs_blk,
                            preferred_element_type=jnp.float32)

    abf_ref[rows, :] = a.astype(jnp.bfloat16)

    acc_ref[rows, :] = jnp.dot(abf_ref[rows, :], hs_ref[...],
                               preferred_element_type=jnp.float32)

    @pl.when(t == _NBLK - 1)
    def _fin():
        out_ref[...] = degb_ref[...] * acc_ref[...] + b_ref[...]


def kernel(x, edge_index, W, b):
    return pl.pallas_call(
        _fused,
        grid=(_NBLK,),
        in_specs=[
            pl.BlockSpec((_BLK, _N), lambda t: (t, 0)),
            pl.BlockSpec((_N, _DIN), lambda t: (0, 0)),
            pl.BlockSpec((_DIN, _DOUT), lambda t: (0, 0)),
            pl.BlockSpec((1, _DOUT), lambda t: (0, 0)),
        ],
        out_specs=pl.BlockSpec((_N, _DOUT), lambda t: (0, 0)),
        out_shape=jax.ShapeDtypeStruct((_N, _DOUT), jnp.float32),
        scratch_shapes=[
            pltpu.VMEM((_N, _N), jnp.bfloat16),
            pltpu.VMEM((_N, _DOUT), jnp.float32),
            pltpu.VMEM((_N, _DOUT), jnp.float32),
            pltpu.VMEM((_N, _DOUT), jnp.bfloat16),
            pltpu.VMEM((_N, _DOUT), jnp.float32),
        ],
    )(edge_index, x, W, b.reshape(1, _DOUT))


# speedup vs baseline: 1.1765x; 1.1765x over previous
import jax
import jax.numpy as jnp
from jax.experimental import pallas as pl
from jax.experimental.pallas import tpu as pltpu

_N = 4096
_DIN = 128
_DOUT = 128
_BLK = 512
_NBLK = _N // _BLK


def _fused(a_ref, x_ref, w_ref, b_ref, out_ref,
           abf_ref, degb_ref, hs_ref, acc_ref):
    t = pl.program_id(0)
    rows = pl.ds(t * _BLK, _BLK)
    cols = pl.ds(t * _BLK, _BLK)

    a = a_ref[...]
    s = jnp.sum(a, axis=1, keepdims=True)
    d = jax.lax.rsqrt(s)
    d = jnp.where(jnp.isinf(d), 0.0, d)
    db = jnp.broadcast_to(d, (_BLK, _DOUT))
    degb_ref[rows, :] = db
    h_blk = jnp.dot(x_ref[rows, :], w_ref[...],
                    preferred_element_type=jnp.float32)
    hs_blk = (db * h_blk).astype(jnp.bfloat16)
    hs_ref[rows, :] = hs_blk
    abf_ref[rows, :] = a.astype(jnp.bfloat16)

    def _col(r, _):
        rr = pl.ds(r * _BLK, _BLK)
        acc_ref[rr, :] += jnp.dot(abf_ref[rr, cols], hs_blk,
                                  preferred_element_type=jnp.float32)
        return 0

    jax.lax.fori_loop(0, t, _col, 0)

    def _row(c, partial):
        cc = pl.ds(c * _BLK, _BLK)
        return partial + jnp.dot(abf_ref[rows, cc], hs_ref[cc, :],
                                 preferred_element_type=jnp.float32)

    acc_ref[rows, :] = jax.lax.fori_loop(
        0, t + 1, _row, jnp.zeros((_BLK, _DOUT), jnp.float32))

    @pl.when(t == _NBLK - 1)
    def _fin():
        out_ref[...] = degb_ref[...] * acc_ref[...] + b_ref[...]


def kernel(x, edge_index, W, b):
    return pl.pallas_call(
        _fused,
        grid=(_NBLK,),
        in_specs=[
            pl.BlockSpec((_BLK, _N), lambda t: (t, 0)),
            pl.BlockSpec((_N, _DIN), lambda t: (0, 0)),
            pl.BlockSpec((_DIN, _DOUT), lambda t: (0, 0)),
            pl.BlockSpec((1, _DOUT), lambda t: (0, 0)),
        ],
        out_specs=pl.BlockSpec((_N, _DOUT), lambda t: (0, 0)),
        out_shape=jax.ShapeDtypeStruct((_N, _DOUT), jnp.float32),
        scratch_shapes=[
            pltpu.VMEM((_N, _N), jnp.bfloat16),
            pltpu.VMEM((_N, _DOUT), jnp.float32),
            pltpu.VMEM((_N, _DOUT), jnp.bfloat16),
            pltpu.VMEM((_N, _DOUT), jnp.float32),
        ],
    )(edge_index, x, W, b.reshape(1, _DOUT))


# speedup vs baseline: 1.2825x; 1.0900x over previous
import jax
import jax.numpy as jnp
from jax.experimental import pallas as pl
from jax.experimental.pallas import tpu as pltpu

_N = 4096
_DIN = 128
_DOUT = 128
_BLK = 512
_NBLK = _N // _BLK


def _fused(a_ref, x_ref, w_ref, b_ref, out_ref,
           abf_ref, degb_ref, hs_ref, acc_ref):
    t = pl.program_id(0)
    rows = pl.ds(t * _BLK, _BLK)
    cols = pl.ds(t * _BLK, _BLK)

    @pl.when(t == 0)
    def _init():
        hs_ref[...] = jnp.zeros((_N, _DOUT), jnp.bfloat16)

    a = a_ref[...]
    s = jnp.sum(a, axis=1, keepdims=True)
    abf_ref[rows, :] = a.astype(jnp.bfloat16)
    d = jax.lax.rsqrt(s)
    d = jnp.where(jnp.isinf(d), 0.0, d)
    db = jnp.broadcast_to(d, (_BLK, _DOUT))
    degb_ref[rows, :] = db
    h_blk = jnp.dot(x_ref[rows, :], w_ref[...],
                    preferred_element_type=jnp.float32)
    hs_blk = (db * h_blk).astype(jnp.bfloat16)
    hs_ref[rows, :] = hs_blk

    for c0 in range(0, _N, 1024):
        acc_ref[c0:c0 + 1024, :] += jnp.dot(
            abf_ref[c0:c0 + 1024, cols], hs_blk,
            preferred_element_type=jnp.float32)

    acc_ref[rows, :] = jnp.dot(abf_ref[rows, :], hs_ref[...],
                               preferred_element_type=jnp.float32)

    @pl.when(t == _NBLK - 1)
    def _fin():
        out_ref[...] = degb_ref[...] * acc_ref[...] + b_ref[...]


def kernel(x, edge_index, W, b):
    return pl.pallas_call(
        _fused,
        grid=(_NBLK,),
        in_specs=[
            pl.BlockSpec((_BLK, _N), lambda t: (t, 0)),
            pl.BlockSpec((_N, _DIN), lambda t: (0, 0)),
            pl.BlockSpec((_DIN, _DOUT), lambda t: (0, 0)),
            pl.BlockSpec((1, _DOUT), lambda t: (0, 0)),
        ],
        out_specs=pl.BlockSpec((_N, _DOUT), lambda t: (0, 0)),
        out_shape=jax.ShapeDtypeStruct((_N, _DOUT), jnp.float32),
        scratch_shapes=[
            pltpu.VMEM((_N, _N), jnp.bfloat16),
            pltpu.VMEM((_N, _DOUT), jnp.float32),
            pltpu.VMEM((_N, _DOUT), jnp.bfloat16),
            pltpu.VMEM((_N, _DOUT), jnp.float32),
        ],
    )(edge_index, x, W, b.reshape(1, _DOUT))


# speedup vs baseline: 1.2939x; 1.0089x over previous
import jax
import jax.numpy as jnp
from jax.experimental import pallas as pl
from jax.experimental.pallas import tpu as pltpu

_N = 4096
_DIN = 128
_DOUT = 128
_BLK = 512
_NBLK = _N // _BLK


def _fused(a_ref, x_ref, w_ref, b_ref, out_ref,
           abf_ref, degb_ref, hs_ref, acc_ref):
    t = pl.program_id(0)
    rows = pl.ds(t * _BLK, _BLK)
    cols = pl.ds(t * _BLK, _BLK)

    @pl.when(t == 0)
    def _init():
        hs_ref[...] = jnp.zeros((_N, _DOUT), jnp.bfloat16)

    a = a_ref[...]
    s = jnp.sum(a, axis=1, keepdims=True)
    abf_ref[rows, :] = a.astype(jnp.bfloat16)
    d = jax.lax.rsqrt(s)
    d = jnp.where(jnp.isinf(d), 0.0, d)
    db = jnp.broadcast_to(d, (_BLK, _DOUT))
    degb_ref[rows, :] = db
    h_blk = jnp.dot(x_ref[rows, :], w_ref[...],
                    preferred_element_type=jnp.float32)
    hs_blk = (db * h_blk).astype(jnp.bfloat16)
    hs_ref[rows, :] = hs_blk

    for c0 in range(0, _N, 2048):
        acc_ref[c0:c0 + 2048, :] += jnp.dot(
            abf_ref[c0:c0 + 2048, cols], hs_blk,
            preferred_element_type=jnp.float32)

    acc_ref[rows, :] = jnp.dot(abf_ref[rows, :], hs_ref[...],
                               preferred_element_type=jnp.float32)

    @pl.when(t == _NBLK - 1)
    def _fin():
        out_ref[...] = degb_ref[...] * acc_ref[...] + b_ref[...]


def kernel(x, edge_index, W, b):
    return pl.pallas_call(
        _fused,
        grid=(_NBLK,),
        in_specs=[
            pl.BlockSpec((_BLK, _N), lambda t: (t, 0)),
            pl.BlockSpec((_N, _DIN), lambda t: (0, 0)),
            pl.BlockSpec((_DIN, _DOUT), lambda t: (0, 0)),
            pl.BlockSpec((1, _DOUT), lambda t: (0, 0)),
        ],
        out_specs=pl.BlockSpec((_N, _DOUT), lambda t: (0, 0)),
        out_shape=jax.ShapeDtypeStruct((_N, _DOUT), jnp.float32),
        scratch_shapes=[
            pltpu.VMEM((_N, _N), jnp.bfloat16),
            pltpu.VMEM((_N, _DOUT), jnp.float32),
            pltpu.VMEM((_N, _DOUT), jnp.bfloat16),
            pltpu.VMEM((_N, _DOUT), jnp.float32),
        ],
    )(edge_index, x, W, b.reshape(1, _DOUT))


# speedup vs baseline: 1.4426x; 1.1149x over previous
import jax
import jax.numpy as jnp
from jax.experimental import pallas as pl
from jax.experimental.pallas import tpu as pltpu

_N = 4096
_DIN = 128
_DOUT = 128
_BLK = 512
_NBLK = _N // _BLK


def _fused(a_ref, x_ref, w_ref, b_ref, out_ref,
           abf_ref, degb_ref, hs_ref, acc_ref):
    t = pl.program_id(0)
    rows = pl.ds(t * _BLK, _BLK)
    cols = pl.ds(t * _BLK, _BLK)

    @pl.when(t == 0)
    def _init():
        hs_ref[...] = jnp.zeros((_N, _DOUT), jnp.bfloat16)

    a = a_ref[...]
    s = jnp.sum(a, axis=1, keepdims=True)
    abf_ref[rows, :] = a.astype(jnp.bfloat16)
    d = jax.lax.rsqrt(s)
    d = jnp.where(jnp.isinf(d), 0.0, d)
    db = jnp.broadcast_to(d, (_BLK, _DOUT))
    degb_ref[rows, :] = db
    h_blk = jnp.dot(x_ref[rows, :], w_ref[...],
                    preferred_element_type=jnp.float32)
    hs_blk = (db * h_blk).astype(jnp.bfloat16)
    hs_ref[rows, :] = hs_blk

    def _col_case(lo, hi, m):
        @pl.when((t >= lo) & (t < hi))
        def _():
            for c0 in range(0, m, 2048):
                mm = min(2048, m - c0)
                acc_ref[c0:c0 + mm, :] += jnp.dot(
                    abf_ref[c0:c0 + mm, cols], hs_blk,
                    preferred_element_type=jnp.float32)

    _col_case(1, 2, 512)
    _col_case(2, 4, 2048)
    _col_case(4, 6, 3072)
    _col_case(6, 8, 4096)

    def _row_case(lo, hi, k):
        @pl.when((t >= lo) & (t < hi))
        def _():
            acc_ref[rows, :] = jnp.dot(abf_ref[rows, 0:k], hs_ref[0:k, :],
                                       preferred_element_type=jnp.float32)

    _row_case(0, 1, 512)
    _row_case(1, 2, 1024)
    _row_case(2, 4, 2048)
    _row_case(4, 6, 3072)
    _row_case(6, 8, 4096)

    @pl.when(t == _NBLK - 1)
    def _fin():
        out_ref[...] = degb_ref[...] * acc_ref[...] + b_ref[...]


def kernel(x, edge_index, W, b):
    return pl.pallas_call(
        _fused,
        grid=(_NBLK,),
        in_specs=[
            pl.BlockSpec((_BLK, _N), lambda t: (t, 0)),
            pl.BlockSpec((_N, _DIN), lambda t: (0, 0)),
            pl.BlockSpec((_DIN, _DOUT), lambda t: (0, 0)),
            pl.BlockSpec((1, _DOUT), lambda t: (0, 0)),
        ],
        out_specs=pl.BlockSpec((_N, _DOUT), lambda t: (0, 0)),
        out_shape=jax.ShapeDtypeStruct((_N, _DOUT), jnp.float32),
        scratch_shapes=[
            pltpu.VMEM((_N, _N), jnp.bfloat16),
            pltpu.VMEM((_N, _DOUT), jnp.float32),
            pltpu.VMEM((_N, _DOUT), jnp.bfloat16),
            pltpu.VMEM((_N, _DOUT), jnp.float32),
        ],
    )(edge_index, x, W, b.reshape(1, _DOUT))
